# baseline (speedup 1.0000x reference)
import jax
import jax.numpy as jnp
from jax import lax
from jax.experimental import pallas as pl
from jax.experimental.pallas import tpu as pltpu
from jax.experimental.pallas import tpu_sc as plsc

N_FACTORS = 64
BATCH = 16384
_LANES = 16
_NW = 32
_BPW = BATCH // _NW
_CHUNKS = _BPW // 128
_GROUPS = _BPW // _LANES


def _sc_kernel(u_hbm, i_hbm, ue_hbm, ie_hbm, ub_hbm, ib_hbm, gb_hbm,
               out_hbm,
               idx_u, idx_i, pu, qi, ubv, ibv, outv, gbv, sem):
    nc = 2
    wid = lax.axis_index("s") * nc + lax.axis_index("c")
    blk = wid * _CHUNKS

    pltpu.sync_copy(u_hbm.at[pl.ds(blk, _CHUNKS), :], idx_u)
    pltpu.sync_copy(i_hbm.at[pl.ds(blk, _CHUNKS), :], idx_i)
    pltpu.sync_copy(gb_hbm, gbv.at[pl.ds(0, 1)])

    copies = []
    for j in range(_CHUNKS):
        rows = pl.ds(j * 128, 128)
        copies.append(pltpu.async_copy(ue_hbm.at[idx_u.at[j]], pu.at[rows], sem))
        copies.append(pltpu.async_copy(ie_hbm.at[idx_i.at[j]], qi.at[rows], sem))
        copies.append(pltpu.async_copy(ub_hbm.at[idx_u.at[j]], ubv.at[rows], sem))
        copies.append(pltpu.async_copy(ib_hbm.at[idx_i.at[j]], ibv.at[rows], sem))
    for c in copies:
        c.wait()

    gb = gbv[pl.ds(0, _LANES)][0]
    lane = lax.iota(jnp.int32, _LANES)

    def group_body(g, carry):
        rows = g * _LANES + lane
        acc = jnp.zeros((_LANES,), jnp.float32)
        for f in range(N_FACTORS):
            col = jnp.full((_LANES,), f, jnp.int32)
            a = plsc.load_gather(pu, [rows, col])
            b = plsc.load_gather(qi, [rows, col])
            acc = acc + a * b
        ub = ubv[pl.ds(g * _LANES, _LANES)]
        ib = ibv[pl.ds(g * _LANES, _LANES)]
        outv[pl.ds(g * _LANES, _LANES)] = acc + ub + ib + gb
        return carry

    lax.fori_loop(0, _GROUPS, group_body, 0)

    pltpu.sync_copy(outv, out_hbm.at[pl.ds(wid * _BPW, _BPW)])


@jax.jit
def _run(u2, i2, user_emb, item_emb, user_bias, item_bias, global_bias):
    mesh = plsc.VectorSubcoreMesh(core_axis_name="c", subcore_axis_name="s")
    return pl.kernel(
        _sc_kernel,
        mesh=mesh,
        out_type=jax.ShapeDtypeStruct((BATCH,), jnp.float32),
        compiler_params=pltpu.CompilerParams(
            needs_layout_passes=False, use_tc_tiling_on_sc=False),
        scratch_types=[
            pltpu.VMEM((_CHUNKS, 128), jnp.int32),
            pltpu.VMEM((_CHUNKS, 128), jnp.int32),
            pltpu.VMEM((_BPW, N_FACTORS), jnp.float32),
            pltpu.VMEM((_BPW, N_FACTORS), jnp.float32),
            pltpu.VMEM((_BPW,), jnp.float32),
            pltpu.VMEM((_BPW,), jnp.float32),
            pltpu.VMEM((_BPW,), jnp.float32),
            pltpu.VMEM((_LANES,), jnp.float32),
            pltpu.SemaphoreType.DMA,
        ],
    )(u2, i2, user_emb, item_emb, user_bias, item_bias, global_bias)


def kernel(u, i, user_emb, item_emb, user_bias, item_bias, global_bias):
    u2 = u.reshape(BATCH // 128, 128)
    i2 = i.reshape(BATCH // 128, 128)
    return _run(u2, i2, user_emb, item_emb, user_bias.reshape(-1),
                item_bias.reshape(-1), global_bias)
